# Initial kernel scaffold; baseline (speedup 1.0000x reference)
import jax
import jax.numpy as jnp
from jax.experimental import pallas as pl


def kernel(nodes, edges, attention, edge_index):
    raise NotImplementedError("write your pallas kernel here")


# speedup vs baseline: 7.3371x; 7.3371x over previous
import functools

import jax
import jax.numpy as jnp
from jax import lax
from jax.experimental import pallas as pl
from jax.experimental.pallas import tpu as pltpu
from jax.experimental.pallas import tpu_sc as plsc

N_CORES = 2
N_SUBCORES = 16
N_WORKERS = N_CORES * N_SUBCORES
LANES = 16
CHUNK = 128
BLK = 16


def _sc_aggregate(n, d, e_pad, recv2d, att2d, edges):
    n_rows = e_pad // CHUNK
    rows_p1 = n_rows // N_SUBCORES
    rows_p2 = n_rows // N_WORKERS
    e_real = edges.shape[0]
    n_per_tile = (n // N_SUBCORES) // 8 * 8
    n_tail = n - n_per_tile * N_SUBCORES
    mesh = plsc.VectorSubcoreMesh(core_axis_name="c", subcore_axis_name="s")

    @functools.partial(
        pl.kernel,
        mesh=mesh,
        out_type=jax.ShapeDtypeStruct((N_CORES, n, d), jnp.float32),
        compiler_params=pltpu.CompilerParams(needs_layout_passes=False),
        scratch_types=[
            pltpu.VMEM((BLK, CHUNK), jnp.int32),
            pltpu.VMEM((BLK, CHUNK), jnp.float32),
            pltpu.VMEM((CHUNK,), jnp.float32),
            pltpu.VMEM((CHUNK,), jnp.float32),
            pltpu.VMEM((n,), jnp.float32),
            pltpu.VMEM((CHUNK, 128), jnp.float32),
            pltpu.VMEM_SHARED((n,), jnp.float32),
            pltpu.VMEM_SHARED((n, 128), jnp.float32),
        ],
    )
    def k(recv_hbm, att_hbm, edges_hbm, out_hbm,
          idx_v, att_v, exp_v, a_v, den_v, ebuf, denom_sh, out_sh):
        cid = lax.axis_index("c")
        sid = lax.axis_index("s")
        wid = sid * N_CORES + cid

        zero16 = jnp.zeros((LANES,), jnp.float32)

        def _zrow(i, _):
            for kk in range(128 // LANES):
                ebuf[i, pl.ds(kk * LANES, LANES)] = zero16
            return 0

        lax.fori_loop(0, CHUNK, _zrow, 0)

        def _zden(i, _):
            den_v[pl.ds(i * LANES, LANES)] = zero16
            return 0

        lax.fori_loop(0, n // LANES, _zden, 0)

        rbase = sid * n_per_tile
        nfull = n_per_tile // CHUNK
        for b in range(nfull):
            pltpu.sync_copy(ebuf, out_sh.at[pl.ds(rbase + b * CHUNK, CHUNK)])
        rem = n_per_tile - nfull * CHUNK
        if rem:
            pltpu.sync_copy(ebuf.at[pl.ds(0, rem)],
                            out_sh.at[pl.ds(rbase + nfull * CHUNK, rem)])

        @pl.when(sid == 0)
        def _():
            pltpu.sync_copy(den_v, denom_sh)
            if n_tail:
                pltpu.sync_copy(ebuf.at[pl.ds(0, n_tail)],
                                out_sh.at[pl.ds(n - n_tail, n_tail)])

        plsc.subcore_barrier()

        p1 = sid * rows_p1

        def _p1_blk(b, _):
            pltpu.sync_copy(recv_hbm.at[pl.ds(p1 + b * BLK, BLK)], idx_v)
            pltpu.sync_copy(att_hbm.at[pl.ds(p1 + b * BLK, BLK)], att_v)

            def _p1(r, _):
                for g in range(CHUNK // LANES):
                    a16 = att_v[r, pl.ds(g * LANES, LANES)]
                    exp_v[pl.ds(g * LANES, LANES)] = jnp.exp(a16)
                pltpu.sync_copy(exp_v, denom_sh.at[idx_v.at[r]], add=True)
                return 0

            lax.fori_loop(0, BLK, _p1, 0)
            return 0

        lax.fori_loop(0, rows_p1 // BLK, _p1_blk, 0)
        plsc.subcore_barrier()

        pltpu.sync_copy(denom_sh, den_v)

        p2 = wid * rows_p2
        ebase0 = wid * (rows_p2 * CHUNK)
        nv = jnp.clip((e_real - ebase0) // CHUNK, 0, rows_p2)

        def _p2_blk(b, _):
            pltpu.sync_copy(recv_hbm.at[pl.ds(p2 + b * BLK, BLK)], idx_v)
            pltpu.sync_copy(att_hbm.at[pl.ds(p2 + b * BLK, BLK)], att_v)
            nvb = jnp.clip(nv - b * BLK, 0, BLK)

            def _p2(c, _):
                pltpu.sync_copy(
                    edges_hbm.at[pl.ds(ebase0 + (b * BLK + c) * CHUNK, CHUNK)],
                    ebuf)
                for g in range(CHUNK // LANES):
                    idx16 = idx_v[c, pl.ds(g * LANES, LANES)]
                    att16 = att_v[c, pl.ds(g * LANES, LANES)]
                    d16 = plsc.load_gather(den_v, [idx16])
                    e16 = jnp.exp(att16)
                    a16 = jnp.where(e16 <= 0.0, 0.0, e16 / d16)
                    a_v[pl.ds(g * LANES, LANES)] = a16

                def _scale(j, _):
                    sp = plsc.load_gather(a_v, [jnp.full((LANES,), j, jnp.int32)])
                    for kk in range(128 // LANES):
                        sl = pl.ds(kk * LANES, LANES)
                        ebuf[j, sl] = ebuf[j, sl] * sp
                    return 0

                lax.fori_loop(0, CHUNK, _scale, 0)
                pltpu.sync_copy(ebuf, out_sh.at[idx_v.at[c]], add=True)
                return 0

            lax.fori_loop(0, nvb, _p2, 0)
            return 0

        lax.fori_loop(0, rows_p2 // BLK, _p2_blk, 0)
        plsc.subcore_barrier()

        pltpu.sync_copy(out_sh.at[pl.ds(rbase, n_per_tile)],
                        out_hbm.at[cid, pl.ds(rbase, n_per_tile)])

        @pl.when(sid == 0)
        def _():
            if n_tail:
                pltpu.sync_copy(out_sh.at[pl.ds(n - n_tail, n_tail)],
                                out_hbm.at[cid, pl.ds(n - n_tail, n_tail)])

    return k(recv2d, att2d, edges)


def _combine(p_ref, o_ref):
    o_ref[...] = p_ref[0] + p_ref[1]


def kernel(nodes, edges, attention, edge_index):
    n = nodes.shape[0]
    e = edges.shape[0]
    d = edges.shape[1]

    recv = edge_index[0].astype(jnp.int32)
    att = attention[:, 0].astype(jnp.float32)

    block = N_WORKERS * CHUNK * 8
    e_pad = ((e + block - 1) // block) * block
    pad = e_pad - e
    recv_p = jnp.concatenate([recv, jnp.zeros((pad,), jnp.int32)])
    att_p = jnp.concatenate([att, jnp.full((pad,), -1e30, jnp.float32)])
    recv2d = recv_p.reshape(-1, CHUNK)
    att2d = att_p.reshape(-1, CHUNK)

    partials = _sc_aggregate(n, d, e_pad, recv2d, att2d, edges)

    nb = 1000
    out = pl.pallas_call(
        _combine,
        grid=(n // nb,),
        in_specs=[pl.BlockSpec((N_CORES, nb, d), lambda i: (0, i, 0))],
        out_specs=pl.BlockSpec((nb, d), lambda i: (i, 0)),
        out_shape=jax.ShapeDtypeStruct((n, d), jnp.float32),
    )(partials)
    return out
